# Initial kernel scaffold; baseline (speedup 1.0000x reference)
import jax
import jax.numpy as jnp
from jax.experimental import pallas as pl


def kernel(feat, shuf_feat, edge_index, diff_edge_index, edge_weight, W1, b1, a1, W2, b2, a2, Wb, bb):
    raise NotImplementedError("write your pallas kernel here")


# speedup vs baseline: 3.6782x; 3.6782x over previous
import functools
import jax
import jax.numpy as jnp
from jax import lax
from jax.experimental import pallas as pl
from jax.experimental.pallas import tpu as pltpu
from jax.experimental.pallas import tpu_sc as plsc

N = 10000
E = 160000
IN_DIM = 256
OUT_DIM = 512

NC = 2
NS = 16
LANES = 16

KB = 128
NBAT = E // KB
NBK = NBAT // NS
NEXTRA = NBAT - NS * NBK

NPAD = 10112
RPT = NPAD // NS

CW = 128
NCH = 1024 // CW


@functools.cache
def _mesh():
    return plsc.VectorSubcoreMesh(core_axis_name="c", subcore_axis_name="s",
                                  num_cores=NC, num_subcores=NS)


@functools.cache
def _get_degree_kernel():
    return functools.partial(
        pl.kernel,
        out_type=jax.ShapeDtypeStruct((2, NPAD, CW), jnp.float32),
        mesh=_mesh(),
        scratch_types=dict(
            hist=pltpu.VMEM_SHARED((NPAD, CW), jnp.float32),
            idx=pltpu.VMEM((KB,), jnp.int32),
            ones=pltpu.VMEM((KB, CW), jnp.float32),
        ),
    )(_degree_body)


def _degree_kernel(ei, zhbm):
    return _get_degree_kernel()(ei, zhbm)


def _degree_body(ei, zhbm, out, hist, idx, ones):
    c = lax.axis_index("c")
    s = lax.axis_index("s")

    def fill(i, _):
        for g in range(CW // LANES):
            ones[i, pl.ds(g * LANES, LANES)] = jnp.ones((LANES,), jnp.float32)
        return 0

    lax.fori_loop(0, KB, fill, 0)

    soff = pl.multiple_of(s * RPT, 8)
    pltpu.sync_copy(zhbm.at[pl.ds(soff, RPT)], hist.at[pl.ds(soff, RPT)])
    plsc.subcore_barrier()

    def do_batch(goff):
        pltpu.sync_copy(ei.at[c].at[pl.ds(goff, KB)], idx)
        pltpu.sync_copy(ones, hist.at[idx], add=True)

    for k in range(NBK):
        do_batch(pl.multiple_of((s + NS * k) * KB, 128))

    for x in range(NEXTRA):
        @pl.when(s == x)
        def _():
            do_batch((NS * NBK + x) * KB)

    plsc.subcore_barrier()

    pltpu.sync_copy(hist.at[pl.ds(soff, RPT)], out.at[c].at[pl.ds(soff, RPT)])


def _mm_body(feat_ref, shuf_ref, w1_ref, w2_ref, dg_ref, tb_ref, tn_ref):
    ch = pl.program_id(1)
    d = lax.rsqrt(jnp.maximum(dg_ref[...], 1.0))

    @pl.when(ch < NCH // 2)
    def _():
        tb_ref[0] = jnp.dot(feat_ref[...], w1_ref[...],
                            preferred_element_type=jnp.float32) * d
        tn_ref[0] = jnp.dot(feat_ref[...], w2_ref[...],
                            preferred_element_type=jnp.float32)

    @pl.when(ch >= NCH // 2)
    def _():
        tb_ref[0] = jnp.dot(shuf_ref[...], w1_ref[...],
                            preferred_element_type=jnp.float32) * d
        tn_ref[0] = jnp.dot(shuf_ref[...], w2_ref[...],
                            preferred_element_type=jnp.float32)


def _make_tables(feat, shuf, W1, W2, dout_col):
    bm = 1000
    grid = (N // bm, NCH)
    return pl.pallas_call(
        _mm_body,
        grid=grid,
        in_specs=[
            pl.BlockSpec((bm, IN_DIM), lambda i, ch: (i, 0)),
            pl.BlockSpec((bm, IN_DIM), lambda i, ch: (i, 0)),
            pl.BlockSpec((IN_DIM, CW), lambda i, ch: (0, ch % (NCH // 2))),
            pl.BlockSpec((IN_DIM, CW), lambda i, ch: (0, ch % (NCH // 2))),
            pl.BlockSpec((bm, 1), lambda i, ch: (i, 0)),
        ],
        out_specs=[
            pl.BlockSpec((1, bm, CW), lambda i, ch: (ch, i, 0)),
            pl.BlockSpec((1, bm, CW), lambda i, ch: (ch, i, 0)),
        ],
        out_shape=[
            jax.ShapeDtypeStruct((NCH, N, CW), jnp.float32),
            jax.ShapeDtypeStruct((NCH, N, CW), jnp.float32),
        ],
    )(feat, shuf, W1, W2, dout_col)


@functools.cache
def _get_agg_kernel():
    return functools.partial(
        pl.kernel,
        out_type=(
            jax.ShapeDtypeStruct((NCH, NPAD, CW), jnp.float32),
            jax.ShapeDtypeStruct((NCH, NPAD, CW), jnp.float32),
        ),
        mesh=_mesh(),
        scratch_types=dict(
            accum=pltpu.VMEM_SHARED((NPAD, CW), jnp.float32),
            sidx0=pltpu.VMEM((KB,), jnp.int32),
            sidx1=pltpu.VMEM((KB,), jnp.int32),
            didx=pltpu.VMEM((KB,), jnp.int32),
            wbuf=pltpu.VMEM((KB,), jnp.float32),
            rows0=pltpu.VMEM((KB, CW), jnp.float32),
            rows1=pltpu.VMEM((KB, CW), jnp.float32),
            sem0=pltpu.SemaphoreType.DMA,
            sem1=pltpu.SemaphoreType.DMA,
        ),
    )(_agg_body)


def _agg_kernel(tb, tn, ei, dei, ew, zhbm):
    return _get_agg_kernel()(tb, tn, ei, dei, ew, zhbm)


def _goff(s, k):
    if isinstance(k, int) and k >= NBK:
        return (NS * NBK + (k - NBK)) * KB
    return pl.multiple_of((s + NS * k) * KB, 128)


def _agg_body(tb, tn, ei, dei, ew, zhbm, aggb, aggn,
              accum, sidx0, sidx1, didx, wbuf, rows0, rows1, sem0, sem1):
    c = lax.axis_index("c")
    s = lax.axis_index("s")
    soff = pl.multiple_of(s * RPT, 8)

    for table_id in range(2):
        edges = ei if table_id == 0 else dei
        agg = aggb if table_id == 0 else aggn
        tbl = tb if table_id == 0 else tn
        use_w = table_id == 1
        sidx = (sidx0, sidx1)
        rows = (rows0, rows1)
        sems = (sem0, sem1)

        for cc in range(NCH // NC):
            ch = c * (NCH // NC) + cc
            pltpu.sync_copy(zhbm.at[pl.ds(soff, RPT)],
                            accum.at[pl.ds(soff, RPT)])
            plsc.subcore_barrier()

            def start_gather(kslot, k):
                pltpu.sync_copy(edges.at[0].at[pl.ds(_goff(s, k), KB)],
                                sidx[kslot])
                pltpu.async_copy(tbl.at[ch].at[sidx[kslot]],
                                 rows[kslot], sems[kslot])

            def finish_batch(kslot, k):
                pltpu.make_async_copy(tbl.at[ch].at[sidx[kslot]],
                                      rows[kslot], sems[kslot]).wait()
                if use_w:
                    pltpu.sync_copy(ew.at[pl.ds(_goff(s, k), KB)], wbuf)
                    rbuf = rows[kslot]

                    def mul16(r16, _):
                        base = pl.multiple_of(r16 * LANES, LANES)
                        wv16 = wbuf[pl.ds(base, LANES)]
                        for j in range(LANES):
                            wvj = jnp.full((LANES,), 1.0, jnp.float32) * wv16[j]
                            for g in range(CW // LANES):
                                sl = pl.ds(g * LANES, LANES)
                                rbuf[base + j, sl] = rbuf[base + j, sl] * wvj
                        return 0

                    lax.fori_loop(0, KB // LANES, mul16, 0)
                pltpu.sync_copy(edges.at[1].at[pl.ds(_goff(s, k), KB)], didx)
                pltpu.sync_copy(rows[kslot], accum.at[didx], add=True)

            start_gather(0, 0)

            def pair_body(t, _):
                k0 = t * 2
                start_gather(1, k0 + 1)
                finish_batch(0, k0)

                @pl.when(t + 1 < NBK // 2)
                def _():
                    start_gather(0, k0 + 2)

                finish_batch(1, k0 + 1)
                return 0

            lax.fori_loop(0, NBK // 2, pair_body, 0)

            for x in range(NEXTRA):
                @pl.when(s == x)
                def _():
                    start_gather(0, NBK + x)
                    finish_batch(0, NBK + x)

            plsc.subcore_barrier()

            pltpu.sync_copy(accum.at[pl.ds(soff, RPT)],
                            agg.at[ch].at[pl.ds(soff, RPT)])
            plsc.subcore_barrier()


def _sums_body(aggb_ref, aggn_ref, din_ref, b1_ref, b2_ref, a1_ref, a2_ref,
               s1_ref, s2_ref):
    i = pl.program_id(0)
    bm = aggb_ref.shape[1]
    din = lax.rsqrt(jnp.maximum(din_ref[...], 1.0))
    a1 = a1_ref[0, 0]
    a2 = a2_ref[0, 0]
    row = lax.broadcasted_iota(jnp.int32, (bm, CW), 0) + i * bm
    valid = row < N
    cs1 = []
    cs2 = []
    for ch in range(OUT_DIM // CW):
        x1 = aggb_ref[ch] * din + b1_ref[ch:ch + 1, :]
        h1 = jnp.where(x1 > 0, x1, a1 * x1)
        x2 = aggn_ref[ch] + b2_ref[ch:ch + 1, :]
        h2 = jnp.where(x2 > 0, x2, a2 * x2)
        h1 = jnp.where(valid, h1, 0.0)
        h2 = jnp.where(valid, h2, 0.0)
        cs1.append(jnp.sum(h1, axis=0, keepdims=True))
        cs2.append(jnp.sum(h2, axis=0, keepdims=True))
    cs1 = jnp.concatenate(cs1, axis=0)
    cs2 = jnp.concatenate(cs2, axis=0)

    @pl.when(i == 0)
    def _():
        s1_ref[...] = cs1
        s2_ref[...] = cs2

    @pl.when(i > 0)
    def _():
        s1_ref[...] = s1_ref[...] + cs1
        s2_ref[...] = s2_ref[...] + cs2


def _col_sums(aggb, aggn, din_col, b1r, b2r, a1s, a2s):
    bm = RPT
    nch_h = OUT_DIM // CW
    return pl.pallas_call(
        _sums_body,
        grid=(NPAD // bm,),
        in_specs=[
            pl.BlockSpec((nch_h, bm, CW), lambda i: (0, i, 0)),
            pl.BlockSpec((nch_h, bm, CW), lambda i: (0, i, 0)),
            pl.BlockSpec((bm, 1), lambda i: (i, 0)),
            pl.BlockSpec((nch_h, CW), lambda i: (0, 0)),
            pl.BlockSpec((nch_h, CW), lambda i: (0, 0)),
            pl.BlockSpec(memory_space=pltpu.SMEM),
            pl.BlockSpec(memory_space=pltpu.SMEM),
        ],
        out_specs=[
            pl.BlockSpec((nch_h, CW), lambda i: (0, 0)),
            pl.BlockSpec((nch_h, CW), lambda i: (0, 0)),
        ],
        out_shape=[
            jax.ShapeDtypeStruct((nch_h, CW), jnp.float32),
            jax.ShapeDtypeStruct((nch_h, CW), jnp.float32),
        ],
    )(aggb, aggn, din_col, b1r, b2r, a1s, a2s)


def _bilinear_body(s1_ref, s2_ref, wbt_ref, v_ref):
    c1 = jax.nn.sigmoid(s1_ref[...] / float(N))
    c2 = jax.nn.sigmoid(s2_ref[...] / float(N))
    v1 = jnp.zeros((1, OUT_DIM), jnp.float32)
    v2 = jnp.zeros((1, OUT_DIM), jnp.float32)
    for ch in range(OUT_DIM // CW):
        v1 = v1 + jnp.dot(c1[ch:ch + 1, :], wbt_ref[ch],
                          preferred_element_type=jnp.float32)
        v2 = v2 + jnp.dot(c2[ch:ch + 1, :], wbt_ref[ch],
                          preferred_element_type=jnp.float32)
    v_ref[0:1, :] = v1
    v_ref[1:2, :] = v2


def _bilinear_vecs(s1, s2, wbt3):
    return pl.pallas_call(
        _bilinear_body,
        out_shape=jax.ShapeDtypeStruct((2, OUT_DIM), jnp.float32),
    )(s1, s2, wbt3)


def _logits_body(aggb_ref, aggn_ref, din_ref, v_ref, b1_ref, b2_ref,
                 a1_ref, a2_ref, bb_ref, out_ref):
    din = lax.rsqrt(jnp.maximum(din_ref[...], 1.0))
    a1 = a1_ref[0, 0]
    a2 = a2_ref[0, 0]
    bb = bb_ref[0, 0]
    hb = jnp.concatenate([aggb_ref[ch] for ch in range(NCH)], axis=1)
    hn = jnp.concatenate([aggn_ref[ch] for ch in range(NCH)], axis=1)
    x1 = hb[:, :OUT_DIM] * din + b1_ref[...]
    h1 = jnp.where(x1 > 0, x1, a1 * x1)
    x3 = hb[:, OUT_DIM:] * din + b1_ref[...]
    h3 = jnp.where(x3 > 0, x3, a1 * x3)
    x2 = hn[:, :OUT_DIM] + b2_ref[...]
    h2 = jnp.where(x2 > 0, x2, a2 * x2)
    x4 = hn[:, OUT_DIM:] + b2_ref[...]
    h4 = jnp.where(x4 > 0, x4, a2 * x4)
    dn = (((1,), (1,)), ((), ()))
    d2 = lax.dot_general(h2, v_ref[...], dn, preferred_element_type=jnp.float32)
    d1 = lax.dot_general(h1, v_ref[...], dn, preferred_element_type=jnp.float32)
    d4 = lax.dot_general(h4, v_ref[...], dn, preferred_element_type=jnp.float32)
    d3 = lax.dot_general(h3, v_ref[...], dn, preferred_element_type=jnp.float32)
    out_ref[...] = jnp.concatenate(
        [d2[:, 0:1], d1[:, 1:2], d4[:, 0:1], d3[:, 1:2]], axis=1) + bb


def _logits(aggb, aggn, din_col, v, b1f, b2f, a1s, a2s, bbs):
    bm = RPT
    return pl.pallas_call(
        _logits_body,
        grid=(NPAD // bm,),
        in_specs=[
            pl.BlockSpec((NCH, bm, CW), lambda i: (0, i, 0)),
            pl.BlockSpec((NCH, bm, CW), lambda i: (0, i, 0)),
            pl.BlockSpec((bm, 1), lambda i: (i, 0)),
            pl.BlockSpec((2, OUT_DIM), lambda i: (0, 0)),
            pl.BlockSpec((1, OUT_DIM), lambda i: (0, 0)),
            pl.BlockSpec((1, OUT_DIM), lambda i: (0, 0)),
            pl.BlockSpec(memory_space=pltpu.SMEM),
            pl.BlockSpec(memory_space=pltpu.SMEM),
            pl.BlockSpec(memory_space=pltpu.SMEM),
        ],
        out_specs=pl.BlockSpec((bm, 4), lambda i: (i, 0)),
        out_shape=jax.ShapeDtypeStruct((NPAD, 4), jnp.float32),
    )(aggb, aggn, din_col, v, b1f, b2f, a1s, a2s, bbs)


def kernel(feat, shuf_feat, edge_index, diff_edge_index, edge_weight,
           W1, b1, a1, W2, b2, a2, Wb, bb):
    zhbm = jnp.zeros((NPAD, CW), jnp.float32)
    deg = _degree_kernel(edge_index, zhbm)
    dout_col = deg[0, :N, 0:1]
    din_col = deg[1, :, 0:1]

    tb, tn = _make_tables(feat, shuf_feat, W1, W2, dout_col)
    aggb, aggn = _agg_kernel(tb, tn, edge_index, diff_edge_index,
                             edge_weight, zhbm)

    b1r = b1.reshape(OUT_DIM // CW, CW)
    b2r = b2.reshape(OUT_DIM // CW, CW)
    a1s = a1.reshape(1, 1)
    a2s = a2.reshape(1, 1)
    bbs = bb.reshape(1, 1)
    s1, s2 = _col_sums(aggb, aggn, din_col, b1r, b2r, a1s, a2s)
    v = _bilinear_vecs(s1, s2, Wb.T.reshape(OUT_DIM // CW, CW, OUT_DIM))
    out = _logits(aggb, aggn, din_col, v, b1.reshape(1, OUT_DIM),
                  b2.reshape(1, OUT_DIM), a1s, a2s, bbs)
    return out[:N].T.reshape(4 * N)


# speedup vs baseline: 3.9120x; 1.0636x over previous
import functools
import jax
import jax.numpy as jnp
from jax import lax
from jax.experimental import pallas as pl
from jax.experimental.pallas import tpu as pltpu
from jax.experimental.pallas import tpu_sc as plsc

N = 10000
E = 160000
IN_DIM = 256
OUT_DIM = 512

NC = 2
NS = 16
LANES = 16

KB = 128
NBAT = E // KB
NBK = NBAT // NS
NEXTRA = NBAT - NS * NBK

NPAD = 10112
RPT = NPAD // NS

CW = 128
NCH = 1024 // CW


@functools.cache
def _mesh():
    return plsc.VectorSubcoreMesh(core_axis_name="c", subcore_axis_name="s",
                                  num_cores=NC, num_subcores=NS)


@functools.cache
def _get_degree_kernel():
    return functools.partial(
        pl.kernel,
        out_type=jax.ShapeDtypeStruct((2, NPAD, CW), jnp.float32),
        mesh=_mesh(),
        scratch_types=dict(
            hist=pltpu.VMEM_SHARED((NPAD, CW), jnp.float32),
            idx=pltpu.VMEM((KB,), jnp.int32),
            ones=pltpu.VMEM((KB, CW), jnp.float32),
        ),
    )(_degree_body)


def _degree_kernel(ei, zhbm):
    return _get_degree_kernel()(ei, zhbm)


def _degree_body(ei, zhbm, out, hist, idx, ones):
    c = lax.axis_index("c")
    s = lax.axis_index("s")

    def fill(i, _):
        for g in range(CW // LANES):
            ones[i, pl.ds(g * LANES, LANES)] = jnp.ones((LANES,), jnp.float32)
        return 0

    lax.fori_loop(0, KB, fill, 0)

    soff = pl.multiple_of(s * RPT, 8)
    pltpu.sync_copy(zhbm.at[pl.ds(soff, RPT)], hist.at[pl.ds(soff, RPT)])
    plsc.subcore_barrier()

    def do_batch(goff):
        pltpu.sync_copy(ei.at[c].at[pl.ds(goff, KB)], idx)
        pltpu.sync_copy(ones, hist.at[idx], add=True)

    for k in range(NBK):
        do_batch(pl.multiple_of((s + NS * k) * KB, 128))

    for x in range(NEXTRA):
        @pl.when(s == x)
        def _():
            do_batch((NS * NBK + x) * KB)

    plsc.subcore_barrier()

    pltpu.sync_copy(hist.at[pl.ds(soff, RPT)], out.at[c].at[pl.ds(soff, RPT)])


def _mm_body(feat_ref, shuf_ref, w1_ref, w2_ref, dg_ref, tb_ref, tn_ref):
    ch = pl.program_id(1)
    d = lax.rsqrt(jnp.maximum(dg_ref[...], 1.0))

    @pl.when(ch < NCH // 2)
    def _():
        tb_ref[0] = jnp.dot(feat_ref[...], w1_ref[...],
                            preferred_element_type=jnp.float32) * d
        tn_ref[0] = jnp.dot(feat_ref[...], w2_ref[...],
                            preferred_element_type=jnp.float32)

    @pl.when(ch >= NCH // 2)
    def _():
        tb_ref[0] = jnp.dot(shuf_ref[...], w1_ref[...],
                            preferred_element_type=jnp.float32) * d
        tn_ref[0] = jnp.dot(shuf_ref[...], w2_ref[...],
                            preferred_element_type=jnp.float32)


def _make_tables(feat, shuf, W1, W2, dout_col):
    bm = 1000
    grid = (N // bm, NCH)
    return pl.pallas_call(
        _mm_body,
        grid=grid,
        in_specs=[
            pl.BlockSpec((bm, IN_DIM), lambda i, ch: (i, 0)),
            pl.BlockSpec((bm, IN_DIM), lambda i, ch: (i, 0)),
            pl.BlockSpec((IN_DIM, CW), lambda i, ch: (0, ch % (NCH // 2))),
            pl.BlockSpec((IN_DIM, CW), lambda i, ch: (0, ch % (NCH // 2))),
            pl.BlockSpec((bm, 1), lambda i, ch: (i, 0)),
        ],
        out_specs=[
            pl.BlockSpec((1, bm, CW), lambda i, ch: (ch, i, 0)),
            pl.BlockSpec((1, bm, CW), lambda i, ch: (ch, i, 0)),
        ],
        out_shape=[
            jax.ShapeDtypeStruct((NCH, N, CW), jnp.float32),
            jax.ShapeDtypeStruct((NCH, N, CW), jnp.float32),
        ],
    )(feat, shuf, W1, W2, dout_col)


@functools.cache
def _get_agg_kernel():
    return functools.partial(
        pl.kernel,
        out_type=(
            jax.ShapeDtypeStruct((NCH, NPAD, CW), jnp.float32),
            jax.ShapeDtypeStruct((NCH, NPAD, CW), jnp.float32),
        ),
        mesh=_mesh(),
        scratch_types=dict(
            accum=pltpu.VMEM_SHARED((NPAD, CW), jnp.float32),
            sidx0=pltpu.VMEM((KB,), jnp.int32),
            sidx1=pltpu.VMEM((KB,), jnp.int32),
            didx0=pltpu.VMEM((KB,), jnp.int32),
            didx1=pltpu.VMEM((KB,), jnp.int32),
            wbuf=pltpu.VMEM((KB,), jnp.float32),
            rows0=pltpu.VMEM((KB, CW), jnp.float32),
            rows1=pltpu.VMEM((KB, CW), jnp.float32),
            gsem0=pltpu.SemaphoreType.DMA,
            gsem1=pltpu.SemaphoreType.DMA,
            ssem0=pltpu.SemaphoreType.DMA,
            ssem1=pltpu.SemaphoreType.DMA,
        ),
    )(_agg_body)


def _agg_kernel(tb, tn, ei, dei, ew, zhbm):
    return _get_agg_kernel()(tb, tn, ei, dei, ew, zhbm)


def _goff(s, k):
    if isinstance(k, int) and k >= NBK:
        return (NS * NBK + (k - NBK)) * KB
    return pl.multiple_of((s + NS * k) * KB, 128)


def _agg_body(tb, tn, ei, dei, ew, zhbm, aggb, aggn,
              accum, sidx0, sidx1, didx0, didx1, wbuf, rows0, rows1,
              gsem0, gsem1, ssem0, ssem1):
    c = lax.axis_index("c")
    s = lax.axis_index("s")
    soff = pl.multiple_of(s * RPT, 8)

    for table_id in range(2):
        edges = ei if table_id == 0 else dei
        agg = aggb if table_id == 0 else aggn
        tbl = tb if table_id == 0 else tn
        use_w = table_id == 1
        sidx = (sidx0, sidx1)
        didx = (didx0, didx1)
        rows = (rows0, rows1)
        gsems = (gsem0, gsem1)
        ssems = (ssem0, ssem1)

        for cc in range(NCH // NC):
            ch = c * (NCH // NC) + cc
            pltpu.sync_copy(zhbm.at[pl.ds(soff, RPT)],
                            accum.at[pl.ds(soff, RPT)])
            plsc.subcore_barrier()

            def start_gather(kslot, k):
                pltpu.sync_copy(edges.at[0].at[pl.ds(_goff(s, k), KB)],
                                sidx[kslot])
                pltpu.sync_copy(edges.at[1].at[pl.ds(_goff(s, k), KB)],
                                didx[kslot])
                pltpu.async_copy(tbl.at[ch].at[sidx[kslot]],
                                 rows[kslot], gsems[kslot])

            def mul_scatter(kslot, k):
                pltpu.make_async_copy(tbl.at[ch].at[sidx[kslot]],
                                      rows[kslot], gsems[kslot]).wait()
                if use_w:
                    pltpu.sync_copy(ew.at[pl.ds(_goff(s, k), KB)], wbuf)
                    rbuf = rows[kslot]

                    def mul16(r16, _):
                        base = pl.multiple_of(r16 * LANES, LANES)
                        wv16 = wbuf[pl.ds(base, LANES)]
                        for j in range(LANES):
                            wvj = jnp.full((LANES,), 1.0, jnp.float32) * wv16[j]
                            for g in range(CW // LANES):
                                sl = pl.ds(g * LANES, LANES)
                                rbuf[base + j, sl] = rbuf[base + j, sl] * wvj
                        return 0

                    lax.fori_loop(0, KB // LANES, mul16, 0)
                pltpu.make_async_copy(rows[kslot], accum.at[didx[kslot]],
                                      ssems[kslot]).start(add=True)

            def wait_scatter(kslot):
                pltpu.make_async_copy(rows[kslot], accum.at[didx[kslot]],
                                      ssems[kslot]).wait()

            start_gather(0, 0)
            start_gather(1, 1)

            def pair_body(t, _):
                k0 = t * 2
                mul_scatter(0, k0)
                mul_scatter(1, k0 + 1)

                @pl.when(t + 1 < NBK // 2)
                def _():
                    wait_scatter(0)
                    start_gather(0, k0 + 2)
                    wait_scatter(1)
                    start_gather(1, k0 + 3)

                return 0

            lax.fori_loop(0, NBK // 2, pair_body, 0)
            wait_scatter(0)
            wait_scatter(1)

            for x in range(NEXTRA):
                @pl.when(s == x)
                def _():
                    start_gather(0, NBK + x)
                    mul_scatter(0, NBK + x)
                    wait_scatter(0)

            plsc.subcore_barrier()

            pltpu.sync_copy(accum.at[pl.ds(soff, RPT)],
                            agg.at[ch].at[pl.ds(soff, RPT)])
            plsc.subcore_barrier()


def _sums_body(aggb_ref, aggn_ref, din_ref, b1_ref, b2_ref, a1_ref, a2_ref,
               s1_ref, s2_ref):
    i = pl.program_id(0)
    bm = aggb_ref.shape[1]
    din = lax.rsqrt(jnp.maximum(din_ref[...], 1.0))
    a1 = a1_ref[0, 0]
    a2 = a2_ref[0, 0]
    row = lax.broadcasted_iota(jnp.int32, (bm, CW), 0) + i * bm
    valid = row < N
    cs1 = []
    cs2 = []
    for ch in range(OUT_DIM // CW):
        x1 = aggb_ref[ch] * din + b1_ref[ch:ch + 1, :]
        h1 = jnp.where(x1 > 0, x1, a1 * x1)
        x2 = aggn_ref[ch] + b2_ref[ch:ch + 1, :]
        h2 = jnp.where(x2 > 0, x2, a2 * x2)
        h1 = jnp.where(valid, h1, 0.0)
        h2 = jnp.where(valid, h2, 0.0)
        cs1.append(jnp.sum(h1, axis=0, keepdims=True))
        cs2.append(jnp.sum(h2, axis=0, keepdims=True))
    cs1 = jnp.concatenate(cs1, axis=0)
    cs2 = jnp.concatenate(cs2, axis=0)

    @pl.when(i == 0)
    def _():
        s1_ref[...] = cs1
        s2_ref[...] = cs2

    @pl.when(i > 0)
    def _():
        s1_ref[...] = s1_ref[...] + cs1
        s2_ref[...] = s2_ref[...] + cs2


def _col_sums(aggb, aggn, din_col, b1r, b2r, a1s, a2s):
    bm = RPT
    nch_h = OUT_DIM // CW
    return pl.pallas_call(
        _sums_body,
        grid=(NPAD // bm,),
        in_specs=[
            pl.BlockSpec((nch_h, bm, CW), lambda i: (0, i, 0)),
            pl.BlockSpec((nch_h, bm, CW), lambda i: (0, i, 0)),
            pl.BlockSpec((bm, 1), lambda i: (i, 0)),
            pl.BlockSpec((nch_h, CW), lambda i: (0, 0)),
            pl.BlockSpec((nch_h, CW), lambda i: (0, 0)),
            pl.BlockSpec(memory_space=pltpu.SMEM),
            pl.BlockSpec(memory_space=pltpu.SMEM),
        ],
        out_specs=[
            pl.BlockSpec((nch_h, CW), lambda i: (0, 0)),
            pl.BlockSpec((nch_h, CW), lambda i: (0, 0)),
        ],
        out_shape=[
            jax.ShapeDtypeStruct((nch_h, CW), jnp.float32),
            jax.ShapeDtypeStruct((nch_h, CW), jnp.float32),
        ],
    )(aggb, aggn, din_col, b1r, b2r, a1s, a2s)


def _bilinear_body(s1_ref, s2_ref, wbt_ref, v_ref):
    c1 = jax.nn.sigmoid(s1_ref[...] / float(N))
    c2 = jax.nn.sigmoid(s2_ref[...] / float(N))
    v1 = jnp.zeros((1, OUT_DIM), jnp.float32)
    v2 = jnp.zeros((1, OUT_DIM), jnp.float32)
    for ch in range(OUT_DIM // CW):
        v1 = v1 + jnp.dot(c1[ch:ch + 1, :], wbt_ref[ch],
                          preferred_element_type=jnp.float32)
        v2 = v2 + jnp.dot(c2[ch:ch + 1, :], wbt_ref[ch],
                          preferred_element_type=jnp.float32)
    v_ref[0:1, :] = v1
    v_ref[1:2, :] = v2


def _bilinear_vecs(s1, s2, wbt3):
    return pl.pallas_call(
        _bilinear_body,
        out_shape=jax.ShapeDtypeStruct((2, OUT_DIM), jnp.float32),
    )(s1, s2, wbt3)


def _logits_body(aggb_ref, aggn_ref, din_ref, v_ref, b1_ref, b2_ref,
                 a1_ref, a2_ref, bb_ref, out_ref):
    din = lax.rsqrt(jnp.maximum(din_ref[...], 1.0))
    a1 = a1_ref[0, 0]
    a2 = a2_ref[0, 0]
    bb = bb_ref[0, 0]
    hb = jnp.concatenate([aggb_ref[ch] for ch in range(NCH)], axis=1)
    hn = jnp.concatenate([aggn_ref[ch] for ch in range(NCH)], axis=1)
    x1 = hb[:, :OUT_DIM] * din + b1_ref[...]
    h1 = jnp.where(x1 > 0, x1, a1 * x1)
    x3 = hb[:, OUT_DIM:] * din + b1_ref[...]
    h3 = jnp.where(x3 > 0, x3, a1 * x3)
    x2 = hn[:, :OUT_DIM] + b2_ref[...]
    h2 = jnp.where(x2 > 0, x2, a2 * x2)
    x4 = hn[:, OUT_DIM:] + b2_ref[...]
    h4 = jnp.where(x4 > 0, x4, a2 * x4)
    dn = (((1,), (1,)), ((), ()))
    d2 = lax.dot_general(h2, v_ref[...], dn, preferred_element_type=jnp.float32)
    d1 = lax.dot_general(h1, v_ref[...], dn, preferred_element_type=jnp.float32)
    d4 = lax.dot_general(h4, v_ref[...], dn, preferred_element_type=jnp.float32)
    d3 = lax.dot_general(h3, v_ref[...], dn, preferred_element_type=jnp.float32)
    out_ref[...] = jnp.concatenate(
        [d2[:, 0:1], d1[:, 1:2], d4[:, 0:1], d3[:, 1:2]], axis=1) + bb


def _logits(aggb, aggn, din_col, v, b1f, b2f, a1s, a2s, bbs):
    bm = RPT
    return pl.pallas_call(
        _logits_body,
        grid=(NPAD // bm,),
        in_specs=[
            pl.BlockSpec((NCH, bm, CW), lambda i: (0, i, 0)),
            pl.BlockSpec((NCH, bm, CW), lambda i: (0, i, 0)),
            pl.BlockSpec((bm, 1), lambda i: (i, 0)),
            pl.BlockSpec((2, OUT_DIM), lambda i: (0, 0)),
            pl.BlockSpec((1, OUT_DIM), lambda i: (0, 0)),
            pl.BlockSpec((1, OUT_DIM), lambda i: (0, 0)),
            pl.BlockSpec(memory_space=pltpu.SMEM),
            pl.BlockSpec(memory_space=pltpu.SMEM),
            pl.BlockSpec(memory_space=pltpu.SMEM),
        ],
        out_specs=pl.BlockSpec((bm, 4), lambda i: (i, 0)),
        out_shape=jax.ShapeDtypeStruct((NPAD, 4), jnp.float32),
    )(aggb, aggn, din_col, v, b1f, b2f, a1s, a2s, bbs)


def kernel(feat, shuf_feat, edge_index, diff_edge_index, edge_weight,
           W1, b1, a1, W2, b2, a2, Wb, bb):
    zhbm = jnp.zeros((NPAD, CW), jnp.float32)
    deg = _degree_kernel(edge_index, zhbm)
    dout_col = deg[0, :N, 0:1]
    din_col = deg[1, :, 0:1]

    tb, tn = _make_tables(feat, shuf_feat, W1, W2, dout_col)
    aggb, aggn = _agg_kernel(tb, tn, edge_index, diff_edge_index,
                             edge_weight, zhbm)

    b1r = b1.reshape(OUT_DIM // CW, CW)
    b2r = b2.reshape(OUT_DIM // CW, CW)
    a1s = a1.reshape(1, 1)
    a2s = a2.reshape(1, 1)
    bbs = bb.reshape(1, 1)
    s1, s2 = _col_sums(aggb, aggn, din_col, b1r, b2r, a1s, a2s)
    v = _bilinear_vecs(s1, s2, Wb.T.reshape(OUT_DIM // CW, CW, OUT_DIM))
    out = _logits(aggb, aggn, din_col, v, b1.reshape(1, OUT_DIM),
                  b2.reshape(1, OUT_DIM), a1s, a2s, bbs)
    return out[:N].T.reshape(4 * N)


# speedup vs baseline: 4.3129x; 1.1025x over previous
import functools
import jax
import jax.numpy as jnp
from jax import lax
from jax.experimental import pallas as pl
from jax.experimental.pallas import tpu as pltpu
from jax.experimental.pallas import tpu_sc as plsc

N = 10000
E = 160000
IN_DIM = 256
OUT_DIM = 512

NC = 2
NS = 16
LANES = 16

KB = 128
NBAT = E // KB
NBK = NBAT // NS
NEXTRA = NBAT - NS * NBK

NPAD = 10112
RPT = NPAD // NS

CW = 128
NCH = 1024 // CW


@functools.cache
def _mesh():
    return plsc.VectorSubcoreMesh(core_axis_name="c", subcore_axis_name="s",
                                  num_cores=NC, num_subcores=NS)


@functools.cache
def _get_degree_kernel():
    return functools.partial(
        pl.kernel,
        out_type=jax.ShapeDtypeStruct((2, NPAD, CW), jnp.float32),
        mesh=_mesh(),
        scratch_types=dict(
            hist=pltpu.VMEM_SHARED((NPAD, CW), jnp.float32),
            idx=pltpu.VMEM((KB,), jnp.int32),
            ones=pltpu.VMEM((KB, CW), jnp.float32),
        ),
    )(_degree_body)


def _degree_kernel(ei, zhbm):
    return _get_degree_kernel()(ei, zhbm)


def _degree_body(ei, zhbm, out, hist, idx, ones):
    c = lax.axis_index("c")
    s = lax.axis_index("s")

    def fill(i, _):
        for g in range(CW // LANES):
            ones[i, pl.ds(g * LANES, LANES)] = jnp.ones((LANES,), jnp.float32)
        return 0

    lax.fori_loop(0, KB, fill, 0)

    soff = pl.multiple_of(s * RPT, 8)
    pltpu.sync_copy(zhbm.at[pl.ds(soff, RPT)], hist.at[pl.ds(soff, RPT)])
    plsc.subcore_barrier()

    def do_batch(goff):
        pltpu.sync_copy(ei.at[c].at[pl.ds(goff, KB)], idx)
        pltpu.sync_copy(ones, hist.at[idx], add=True)

    for k in range(NBK):
        do_batch(pl.multiple_of((s + NS * k) * KB, 128))

    for x in range(NEXTRA):
        @pl.when(s == x)
        def _():
            do_batch((NS * NBK + x) * KB)

    plsc.subcore_barrier()

    pltpu.sync_copy(hist.at[pl.ds(soff, RPT)], out.at[c].at[pl.ds(soff, RPT)])


def _mm_body(feat_ref, shuf_ref, w1_ref, w2_ref, dg_ref, tb_ref, tn_ref):
    ch = pl.program_id(1)
    d = lax.rsqrt(jnp.maximum(dg_ref[...], 1.0))

    @pl.when(ch < NCH // 2)
    def _():
        tb_ref[0] = jnp.dot(feat_ref[...], w1_ref[...],
                            preferred_element_type=jnp.float32) * d
        tn_ref[0] = jnp.dot(feat_ref[...], w2_ref[...],
                            preferred_element_type=jnp.float32)

    @pl.when(ch >= NCH // 2)
    def _():
        tb_ref[0] = jnp.dot(shuf_ref[...], w1_ref[...],
                            preferred_element_type=jnp.float32) * d
        tn_ref[0] = jnp.dot(shuf_ref[...], w2_ref[...],
                            preferred_element_type=jnp.float32)


def _make_tables(feat, shuf, W1, W2, dout_col):
    bm = 1000
    grid = (N // bm, NCH)
    return pl.pallas_call(
        _mm_body,
        grid=grid,
        in_specs=[
            pl.BlockSpec((bm, IN_DIM), lambda i, ch: (i, 0)),
            pl.BlockSpec((bm, IN_DIM), lambda i, ch: (i, 0)),
            pl.BlockSpec((IN_DIM, CW), lambda i, ch: (0, ch % (NCH // 2))),
            pl.BlockSpec((IN_DIM, CW), lambda i, ch: (0, ch % (NCH // 2))),
            pl.BlockSpec((bm, 1), lambda i, ch: (i, 0)),
        ],
        out_specs=[
            pl.BlockSpec((1, bm, CW), lambda i, ch: (ch, i, 0)),
            pl.BlockSpec((1, bm, CW), lambda i, ch: (ch, i, 0)),
        ],
        out_shape=[
            jax.ShapeDtypeStruct((NCH, N, CW), jnp.float32),
            jax.ShapeDtypeStruct((NCH, N, CW), jnp.float32),
        ],
    )(feat, shuf, W1, W2, dout_col)


@functools.cache
def _get_agg_kernel():
    return functools.partial(
        pl.kernel,
        out_type=(
            jax.ShapeDtypeStruct((NCH, NPAD, CW), jnp.float32),
            jax.ShapeDtypeStruct((NCH, NPAD, CW), jnp.float32),
        ),
        mesh=_mesh(),
        scratch_types=dict(
            accum=pltpu.VMEM_SHARED((NPAD, CW), jnp.float32),
            sd0=pltpu.VMEM((2 * KB,), jnp.int32),
            sd1=pltpu.VMEM((2 * KB,), jnp.int32),
            didx0=pltpu.VMEM((KB,), jnp.int32),
            didx1=pltpu.VMEM((KB,), jnp.int32),
            wbuf=pltpu.VMEM((KB,), jnp.float32),
            rows0=pltpu.VMEM((KB, CW), jnp.float32),
            rows1=pltpu.VMEM((KB, CW), jnp.float32),
            gsem0=pltpu.SemaphoreType.DMA,
            gsem1=pltpu.SemaphoreType.DMA,
            ssem0=pltpu.SemaphoreType.DMA,
            ssem1=pltpu.SemaphoreType.DMA,
        ),
    )(_agg_body)


def _agg_kernel(tb, tn, eicat, deicat, ew, zhbm):
    return _get_agg_kernel()(tb, tn, eicat, deicat, ew, zhbm)


def _goff(s, k):
    if isinstance(k, int) and k >= NBK:
        return (NS * NBK + (k - NBK)) * KB
    return pl.multiple_of((s + NS * k) * KB, 128)


def _goff2(s, k):
    if isinstance(k, int) and k >= NBK:
        return (NS * NBK + (k - NBK)) * 2 * KB
    return pl.multiple_of((s + NS * k) * 2 * KB, 256)


def _agg_body(tb, tn, eicat, deicat, ew, zhbm, aggb, aggn,
              accum, sd0, sd1, didx0, didx1, wbuf, rows0, rows1,
              gsem0, gsem1, ssem0, ssem1):
    c = lax.axis_index("c")
    s = lax.axis_index("s")
    soff = pl.multiple_of(s * RPT, 8)

    for table_id in range(2):
        edges = eicat if table_id == 0 else deicat
        agg = aggb if table_id == 0 else aggn
        tbl = tb if table_id == 0 else tn
        use_w = table_id == 1
        sd = (sd0, sd1)
        didx = (didx0, didx1)
        rows = (rows0, rows1)
        gsems = (gsem0, gsem1)
        ssems = (ssem0, ssem1)

        for cc in range(NCH // NC):
            ch = c * (NCH // NC) + cc
            pltpu.sync_copy(zhbm.at[pl.ds(soff, RPT)],
                            accum.at[pl.ds(soff, RPT)])
            plsc.subcore_barrier()

            def start_gather(kslot, k):
                pltpu.sync_copy(edges.at[pl.ds(_goff2(s, k), 2 * KB)],
                                sd[kslot])
                pltpu.async_copy(tbl.at[ch].at[sd[kslot].at[pl.ds(0, KB)]],
                                 rows[kslot], gsems[kslot])
                for j in range(KB // LANES):
                    sl = pl.ds(j * LANES, LANES)
                    didx[kslot][sl] = sd[kslot][pl.ds(KB + j * LANES, LANES)]

            def mul_scatter(kslot, k):
                pltpu.make_async_copy(tbl.at[ch].at[sd[kslot].at[pl.ds(0, KB)]],
                                      rows[kslot], gsems[kslot]).wait()
                if use_w:
                    pltpu.sync_copy(ew.at[pl.ds(_goff(s, k), KB)], wbuf)
                    rbuf = rows[kslot]
                    wb = wbuf

                    def mul16(r16, _):
                        base = pl.multiple_of(r16 * LANES, LANES)
                        wv16 = wb[pl.ds(base, LANES)]
                        for j in range(LANES):
                            wvj = jnp.full((LANES,), 1.0, jnp.float32) * wv16[j]
                            for g in range(CW // LANES):
                                sl = pl.ds(g * LANES, LANES)
                                rbuf[base + j, sl] = rbuf[base + j, sl] * wvj
                        return 0

                    lax.fori_loop(0, KB // LANES, mul16, 0)
                pltpu.make_async_copy(rows[kslot], accum.at[didx[kslot]],
                                      ssems[kslot]).start(add=True)

            def wait_scatter(kslot):
                pltpu.make_async_copy(rows[kslot], accum.at[didx[kslot]],
                                      ssems[kslot]).wait()

            start_gather(0, 0)
            start_gather(1, 1)

            def pair_body(t, _):
                k0 = t * 2
                mul_scatter(0, k0)
                mul_scatter(1, k0 + 1)

                @pl.when(t + 1 < NBK // 2)
                def _():
                    wait_scatter(0)
                    start_gather(0, k0 + 2)
                    wait_scatter(1)
                    start_gather(1, k0 + 3)

                return 0

            lax.fori_loop(0, NBK // 2, pair_body, 0)
            wait_scatter(0)
            wait_scatter(1)

            for x in range(NEXTRA):
                @pl.when(s == x)
                def _():
                    start_gather(0, NBK + x)
                    mul_scatter(0, NBK + x)
                    wait_scatter(0)

            plsc.subcore_barrier()

            pltpu.sync_copy(accum.at[pl.ds(soff, RPT)],
                            agg.at[ch].at[pl.ds(soff, RPT)])
            plsc.subcore_barrier()


def _sums_body(aggb_ref, aggn_ref, din_ref, b1_ref, b2_ref, a1_ref, a2_ref,
               s1_ref, s2_ref):
    i = pl.program_id(0)
    bm = aggb_ref.shape[1]
    din = lax.rsqrt(jnp.maximum(din_ref[...], 1.0))
    a1 = a1_ref[0, 0]
    a2 = a2_ref[0, 0]
    row = lax.broadcasted_iota(jnp.int32, (bm, CW), 0) + i * bm
    valid = row < N
    cs1 = []
    cs2 = []
    for ch in range(OUT_DIM // CW):
        x1 = aggb_ref[ch] * din + b1_ref[ch:ch + 1, :]
        h1 = jnp.where(x1 > 0, x1, a1 * x1)
        x2 = aggn_ref[ch] + b2_ref[ch:ch + 1, :]
        h2 = jnp.where(x2 > 0, x2, a2 * x2)
        h1 = jnp.where(valid, h1, 0.0)
        h2 = jnp.where(valid, h2, 0.0)
        cs1.append(jnp.sum(h1, axis=0, keepdims=True))
        cs2.append(jnp.sum(h2, axis=0, keepdims=True))
    cs1 = jnp.concatenate(cs1, axis=0)
    cs2 = jnp.concatenate(cs2, axis=0)

    @pl.when(i == 0)
    def _():
        s1_ref[...] = cs1
        s2_ref[...] = cs2

    @pl.when(i > 0)
    def _():
        s1_ref[...] = s1_ref[...] + cs1
        s2_ref[...] = s2_ref[...] + cs2


def _col_sums(aggb, aggn, din_col, b1r, b2r, a1s, a2s):
    bm = RPT
    nch_h = OUT_DIM // CW
    return pl.pallas_call(
        _sums_body,
        grid=(NPAD // bm,),
        in_specs=[
            pl.BlockSpec((nch_h, bm, CW), lambda i: (0, i, 0)),
            pl.BlockSpec((nch_h, bm, CW), lambda i: (0, i, 0)),
            pl.BlockSpec((bm, 1), lambda i: (i, 0)),
            pl.BlockSpec((nch_h, CW), lambda i: (0, 0)),
            pl.BlockSpec((nch_h, CW), lambda i: (0, 0)),
            pl.BlockSpec(memory_space=pltpu.SMEM),
            pl.BlockSpec(memory_space=pltpu.SMEM),
        ],
        out_specs=[
            pl.BlockSpec((nch_h, CW), lambda i: (0, 0)),
            pl.BlockSpec((nch_h, CW), lambda i: (0, 0)),
        ],
        out_shape=[
            jax.ShapeDtypeStruct((nch_h, CW), jnp.float32),
            jax.ShapeDtypeStruct((nch_h, CW), jnp.float32),
        ],
    )(aggb, aggn, din_col, b1r, b2r, a1s, a2s)


def _bilinear_body(s1_ref, s2_ref, wbt_ref, v_ref):
    c1 = jax.nn.sigmoid(s1_ref[...] / float(N))
    c2 = jax.nn.sigmoid(s2_ref[...] / float(N))
    v1 = jnp.zeros((1, OUT_DIM), jnp.float32)
    v2 = jnp.zeros((1, OUT_DIM), jnp.float32)
    for ch in range(OUT_DIM // CW):
        v1 = v1 + jnp.dot(c1[ch:ch + 1, :], wbt_ref[ch],
                          preferred_element_type=jnp.float32)
        v2 = v2 + jnp.dot(c2[ch:ch + 1, :], wbt_ref[ch],
                          preferred_element_type=jnp.float32)
    v_ref[0:1, :] = v1
    v_ref[1:2, :] = v2


def _bilinear_vecs(s1, s2, wbt3):
    return pl.pallas_call(
        _bilinear_body,
        out_shape=jax.ShapeDtypeStruct((2, OUT_DIM), jnp.float32),
    )(s1, s2, wbt3)


def _logits_body(aggb_ref, aggn_ref, din_ref, v_ref, b1_ref, b2_ref,
                 a1_ref, a2_ref, bb_ref, out_ref):
    din = lax.rsqrt(jnp.maximum(din_ref[...], 1.0))
    a1 = a1_ref[0, 0]
    a2 = a2_ref[0, 0]
    bb = bb_ref[0, 0]
    hb = jnp.concatenate([aggb_ref[ch] for ch in range(NCH)], axis=1)
    hn = jnp.concatenate([aggn_ref[ch] for ch in range(NCH)], axis=1)
    x1 = hb[:, :OUT_DIM] * din + b1_ref[...]
    h1 = jnp.where(x1 > 0, x1, a1 * x1)
    x3 = hb[:, OUT_DIM:] * din + b1_ref[...]
    h3 = jnp.where(x3 > 0, x3, a1 * x3)
    x2 = hn[:, :OUT_DIM] + b2_ref[...]
    h2 = jnp.where(x2 > 0, x2, a2 * x2)
    x4 = hn[:, OUT_DIM:] + b2_ref[...]
    h4 = jnp.where(x4 > 0, x4, a2 * x4)
    dn = (((1,), (1,)), ((), ()))
    d2 = lax.dot_general(h2, v_ref[...], dn, preferred_element_type=jnp.float32)
    d1 = lax.dot_general(h1, v_ref[...], dn, preferred_element_type=jnp.float32)
    d4 = lax.dot_general(h4, v_ref[...], dn, preferred_element_type=jnp.float32)
    d3 = lax.dot_general(h3, v_ref[...], dn, preferred_element_type=jnp.float32)
    out_ref[...] = jnp.concatenate(
        [d2[:, 0:1], d1[:, 1:2], d4[:, 0:1], d3[:, 1:2]], axis=1) + bb


def _logits(aggb, aggn, din_col, v, b1f, b2f, a1s, a2s, bbs):
    bm = RPT
    return pl.pallas_call(
        _logits_body,
        grid=(NPAD // bm,),
        in_specs=[
            pl.BlockSpec((NCH, bm, CW), lambda i: (0, i, 0)),
            pl.BlockSpec((NCH, bm, CW), lambda i: (0, i, 0)),
            pl.BlockSpec((bm, 1), lambda i: (i, 0)),
            pl.BlockSpec((2, OUT_DIM), lambda i: (0, 0)),
            pl.BlockSpec((1, OUT_DIM), lambda i: (0, 0)),
            pl.BlockSpec((1, OUT_DIM), lambda i: (0, 0)),
            pl.BlockSpec(memory_space=pltpu.SMEM),
            pl.BlockSpec(memory_space=pltpu.SMEM),
            pl.BlockSpec(memory_space=pltpu.SMEM),
        ],
        out_specs=pl.BlockSpec((bm, 4), lambda i: (i, 0)),
        out_shape=jax.ShapeDtypeStruct((NPAD, 4), jnp.float32),
    )(aggb, aggn, din_col, v, b1f, b2f, a1s, a2s, bbs)


def kernel(feat, shuf_feat, edge_index, diff_edge_index, edge_weight,
           W1, b1, a1, W2, b2, a2, Wb, bb):
    zhbm = jnp.zeros((NPAD, CW), jnp.float32)
    deg = _degree_kernel(edge_index, zhbm)
    dout_col = deg[0, :N, 0:1]
    din_col = deg[1, :, 0:1]

    tb, tn = _make_tables(feat, shuf_feat, W1, W2, dout_col)
    eicat = jnp.transpose(edge_index.reshape(2, NBAT, KB),
                          (1, 0, 2)).reshape(NBAT * 2 * KB)
    deicat = jnp.transpose(diff_edge_index.reshape(2, NBAT, KB),
                           (1, 0, 2)).reshape(NBAT * 2 * KB)
    aggb, aggn = _agg_kernel(tb, tn, eicat, deicat, edge_weight, zhbm)

    b1r = b1.reshape(OUT_DIM // CW, CW)
    b2r = b2.reshape(OUT_DIM // CW, CW)
    a1s = a1.reshape(1, 1)
    a2s = a2.reshape(1, 1)
    bbs = bb.reshape(1, 1)
    s1, s2 = _col_sums(aggb, aggn, din_col, b1r, b2r, a1s, a2s)
    v = _bilinear_vecs(s1, s2, Wb.T.reshape(OUT_DIM // CW, CW, OUT_DIM))
    out = _logits(aggb, aggn, din_col, v, b1.reshape(1, OUT_DIM),
                  b2.reshape(1, OUT_DIM), a1s, a2s, bbs)
    return out[:N].T.reshape(4 * N)


# speedup vs baseline: 4.4653x; 1.0353x over previous
import functools
import jax
import jax.numpy as jnp
from jax import lax
from jax.experimental import pallas as pl
from jax.experimental.pallas import tpu as pltpu
from jax.experimental.pallas import tpu_sc as plsc

N = 10000
E = 160000
IN_DIM = 256
OUT_DIM = 512

NC = 2
NS = 16
LANES = 16

KB = 128
NBAT = E // KB
NBK = NBAT // NS
NEXTRA = NBAT - NS * NBK

NPAD = 10112
RPT = NPAD // NS

CW = 128
NCH = 1024 // CW


@functools.cache
def _mesh():
    return plsc.VectorSubcoreMesh(core_axis_name="c", subcore_axis_name="s",
                                  num_cores=NC, num_subcores=NS)


@functools.cache
def _get_degree_kernel():
    return functools.partial(
        pl.kernel,
        out_type=jax.ShapeDtypeStruct((2, NPAD, CW), jnp.float32),
        mesh=_mesh(),
        scratch_types=dict(
            hist=pltpu.VMEM_SHARED((NPAD, CW), jnp.float32),
            idx=pltpu.VMEM((KB,), jnp.int32),
            ones=pltpu.VMEM((KB, CW), jnp.float32),
        ),
    )(_degree_body)


def _degree_kernel(ei, zhbm):
    return _get_degree_kernel()(ei, zhbm)


def _degree_body(ei, zhbm, out, hist, idx, ones):
    c = lax.axis_index("c")
    s = lax.axis_index("s")

    def fill(i, _):
        for g in range(CW // LANES):
            ones[i, pl.ds(g * LANES, LANES)] = jnp.ones((LANES,), jnp.float32)
        return 0

    lax.fori_loop(0, KB, fill, 0)

    soff = pl.multiple_of(s * RPT, 8)
    pltpu.sync_copy(zhbm.at[pl.ds(soff, RPT)], hist.at[pl.ds(soff, RPT)])
    plsc.subcore_barrier()

    def do_batch(goff):
        pltpu.sync_copy(ei.at[c].at[pl.ds(goff, KB)], idx)
        pltpu.sync_copy(ones, hist.at[idx], add=True)

    for k in range(NBK):
        do_batch(pl.multiple_of((s + NS * k) * KB, 128))

    for x in range(NEXTRA):
        @pl.when(s == x)
        def _():
            do_batch((NS * NBK + x) * KB)

    plsc.subcore_barrier()

    pltpu.sync_copy(hist.at[pl.ds(soff, RPT)], out.at[c].at[pl.ds(soff, RPT)])


def _mm_body(feat_ref, shuf_ref, w1_ref, w2_ref, dg_ref, tb_ref, tn_ref):
    ch = pl.program_id(1)
    d = lax.rsqrt(jnp.maximum(dg_ref[...], 1.0))

    @pl.when(ch < NCH // 2)
    def _():
        tb_ref[0] = jnp.dot(feat_ref[...], w1_ref[...],
                            preferred_element_type=jnp.float32) * d
        tn_ref[0] = jnp.dot(feat_ref[...], w2_ref[...],
                            preferred_element_type=jnp.float32)

    @pl.when(ch >= NCH // 2)
    def _():
        tb_ref[0] = jnp.dot(shuf_ref[...], w1_ref[...],
                            preferred_element_type=jnp.float32) * d
        tn_ref[0] = jnp.dot(shuf_ref[...], w2_ref[...],
                            preferred_element_type=jnp.float32)


def _make_tables(feat, shuf, W1, W2, dout_col):
    bm = 1000
    grid = (N // bm, NCH)
    return pl.pallas_call(
        _mm_body,
        grid=grid,
        in_specs=[
            pl.BlockSpec((bm, IN_DIM), lambda i, ch: (i, 0)),
            pl.BlockSpec((bm, IN_DIM), lambda i, ch: (i, 0)),
            pl.BlockSpec((IN_DIM, CW), lambda i, ch: (0, ch % (NCH // 2))),
            pl.BlockSpec((IN_DIM, CW), lambda i, ch: (0, ch % (NCH // 2))),
            pl.BlockSpec((bm, 1), lambda i, ch: (i, 0)),
        ],
        out_specs=[
            pl.BlockSpec((1, bm, CW), lambda i, ch: (ch, i, 0)),
            pl.BlockSpec((1, bm, CW), lambda i, ch: (ch, i, 0)),
        ],
        out_shape=[
            jax.ShapeDtypeStruct((NCH, N, CW), jnp.float32),
            jax.ShapeDtypeStruct((NCH, N, CW), jnp.float32),
        ],
    )(feat, shuf, W1, W2, dout_col)


@functools.cache
def _get_agg_kernel():
    return functools.partial(
        pl.kernel,
        out_type=(
            jax.ShapeDtypeStruct((NCH, NPAD, CW), jnp.float32),
            jax.ShapeDtypeStruct((NCH, NPAD, CW), jnp.float32),
        ),
        mesh=_mesh(),
        scratch_types=dict(
            accum=pltpu.VMEM_SHARED((NPAD, CW), jnp.float32),
            sd0=pltpu.VMEM((2 * KB,), jnp.int32),
            sd1=pltpu.VMEM((2 * KB,), jnp.int32),
            didx0=pltpu.VMEM((KB,), jnp.int32),
            didx1=pltpu.VMEM((KB,), jnp.int32),
            wbuf0=pltpu.VMEM((KB,), jnp.float32),
            wbuf1=pltpu.VMEM((KB,), jnp.float32),
            rows0=pltpu.VMEM((KB, CW), jnp.float32),
            rows1=pltpu.VMEM((KB, CW), jnp.float32),
            gsem0=pltpu.SemaphoreType.DMA,
            gsem1=pltpu.SemaphoreType.DMA,
            ssem0=pltpu.SemaphoreType.DMA,
            ssem1=pltpu.SemaphoreType.DMA,
        ),
    )(_agg_body)


def _agg_kernel(tb, tn, eicat, deicat, ew, zhbm):
    return _get_agg_kernel()(tb, tn, eicat, deicat, ew, zhbm)


def _goff(s, k):
    if isinstance(k, int) and k >= NBK:
        return (NS * NBK + (k - NBK)) * KB
    return pl.multiple_of((s + NS * k) * KB, 128)


def _goff2(s, k):
    if isinstance(k, int) and k >= NBK:
        return (NS * NBK + (k - NBK)) * 2 * KB
    return pl.multiple_of((s + NS * k) * 2 * KB, 256)


def _agg_body(tb, tn, eicat, deicat, ew, zhbm, aggb, aggn,
              accum, sd0, sd1, didx0, didx1, wbuf0, wbuf1, rows0, rows1,
              gsem0, gsem1, ssem0, ssem1):
    c = lax.axis_index("c")
    s = lax.axis_index("s")
    soff = pl.multiple_of(s * RPT, 8)

    for table_id in range(2):
        edges = eicat if table_id == 0 else deicat
        agg = aggb if table_id == 0 else aggn
        tbl = tb if table_id == 0 else tn
        use_w = table_id == 1
        sd = (sd0, sd1)
        didx = (didx0, didx1)
        wbuf = (wbuf0, wbuf1)
        rows = (rows0, rows1)
        gsems = (gsem0, gsem1)
        ssems = (ssem0, ssem1)

        for cc in range(NCH // NC):
            ch = c * (NCH // NC) + cc
            pltpu.sync_copy(zhbm.at[pl.ds(soff, RPT)],
                            accum.at[pl.ds(soff, RPT)])
            plsc.subcore_barrier()

            def start_gather(kslot, k):
                pltpu.sync_copy(edges.at[pl.ds(_goff2(s, k), 2 * KB)],
                                sd[kslot])
                pltpu.async_copy(tbl.at[ch].at[sd[kslot].at[pl.ds(0, KB)]],
                                 rows[kslot], gsems[kslot])
                if use_w:
                    pltpu.sync_copy(ew.at[pl.ds(_goff(s, k), KB)],
                                    wbuf[kslot])
                for j in range(KB // LANES):
                    sl = pl.ds(j * LANES, LANES)
                    didx[kslot][sl] = sd[kslot][pl.ds(KB + j * LANES, LANES)]

            def mul_scatter(kslot, k):
                pltpu.make_async_copy(tbl.at[ch].at[sd[kslot].at[pl.ds(0, KB)]],
                                      rows[kslot], gsems[kslot]).wait()
                if use_w:
                    rbuf = rows[kslot]
                    wb = wbuf[kslot]

                    def mul16(r16, _):
                        base = pl.multiple_of(r16 * LANES, LANES)
                        wv16 = wb[pl.ds(base, LANES)]
                        for j in range(LANES):
                            wvj = jnp.full((LANES,), 1.0, jnp.float32) * wv16[j]
                            for g in range(CW // LANES):
                                sl = pl.ds(g * LANES, LANES)
                                rbuf[base + j, sl] = rbuf[base + j, sl] * wvj
                        return 0

                    lax.fori_loop(0, KB // LANES, mul16, 0)
                pltpu.make_async_copy(rows[kslot], accum.at[didx[kslot]],
                                      ssems[kslot]).start(add=True)

            def wait_scatter(kslot):
                pltpu.make_async_copy(rows[kslot], accum.at[didx[kslot]],
                                      ssems[kslot]).wait()

            start_gather(0, 0)
            start_gather(1, 1)

            def pair_body(t, _):
                k0 = t * 2
                mul_scatter(0, k0)
                mul_scatter(1, k0 + 1)

                @pl.when(t + 1 < NBK // 2)
                def _():
                    wait_scatter(0)
                    start_gather(0, k0 + 2)
                    wait_scatter(1)
                    start_gather(1, k0 + 3)

                return 0

            lax.fori_loop(0, NBK // 2, pair_body, 0)
            wait_scatter(0)
            wait_scatter(1)

            for x in range(NEXTRA):
                @pl.when(s == x)
                def _():
                    start_gather(0, NBK + x)
                    mul_scatter(0, NBK + x)
                    wait_scatter(0)

            plsc.subcore_barrier()

            pltpu.sync_copy(accum.at[pl.ds(soff, RPT)],
                            agg.at[ch].at[pl.ds(soff, RPT)])
            plsc.subcore_barrier()


def _sums_body(aggb_ref, aggn_ref, din_ref, b1_ref, b2_ref, a1_ref, a2_ref,
               s1_ref, s2_ref):
    i = pl.program_id(0)
    bm = aggb_ref.shape[1]
    din = lax.rsqrt(jnp.maximum(din_ref[...], 1.0))
    a1 = a1_ref[0, 0]
    a2 = a2_ref[0, 0]
    row = lax.broadcasted_iota(jnp.int32, (bm, CW), 0) + i * bm
    valid = row < N
    cs1 = []
    cs2 = []
    for ch in range(OUT_DIM // CW):
        x1 = aggb_ref[ch] * din + b1_ref[ch:ch + 1, :]
        h1 = jnp.where(x1 > 0, x1, a1 * x1)
        x2 = aggn_ref[ch] + b2_ref[ch:ch + 1, :]
        h2 = jnp.where(x2 > 0, x2, a2 * x2)
        h1 = jnp.where(valid, h1, 0.0)
        h2 = jnp.where(valid, h2, 0.0)
        cs1.append(jnp.sum(h1, axis=0, keepdims=True))
        cs2.append(jnp.sum(h2, axis=0, keepdims=True))
    cs1 = jnp.concatenate(cs1, axis=0)
    cs2 = jnp.concatenate(cs2, axis=0)

    @pl.when(i == 0)
    def _():
        s1_ref[...] = cs1
        s2_ref[...] = cs2

    @pl.when(i > 0)
    def _():
        s1_ref[...] = s1_ref[...] + cs1
        s2_ref[...] = s2_ref[...] + cs2


def _col_sums(aggb, aggn, din_col, b1r, b2r, a1s, a2s):
    bm = RPT
    nch_h = OUT_DIM // CW
    return pl.pallas_call(
        _sums_body,
        grid=(NPAD // bm,),
        in_specs=[
            pl.BlockSpec((nch_h, bm, CW), lambda i: (0, i, 0)),
            pl.BlockSpec((nch_h, bm, CW), lambda i: (0, i, 0)),
            pl.BlockSpec((bm, 1), lambda i: (i, 0)),
            pl.BlockSpec((nch_h, CW), lambda i: (0, 0)),
            pl.BlockSpec((nch_h, CW), lambda i: (0, 0)),
            pl.BlockSpec(memory_space=pltpu.SMEM),
            pl.BlockSpec(memory_space=pltpu.SMEM),
        ],
        out_specs=[
            pl.BlockSpec((nch_h, CW), lambda i: (0, 0)),
            pl.BlockSpec((nch_h, CW), lambda i: (0, 0)),
        ],
        out_shape=[
            jax.ShapeDtypeStruct((nch_h, CW), jnp.float32),
            jax.ShapeDtypeStruct((nch_h, CW), jnp.float32),
        ],
    )(aggb, aggn, din_col, b1r, b2r, a1s, a2s)


def _bilinear_body(s1_ref, s2_ref, wbt_ref, v_ref):
    c1 = jax.nn.sigmoid(s1_ref[...] / float(N))
    c2 = jax.nn.sigmoid(s2_ref[...] / float(N))
    v1 = jnp.zeros((1, OUT_DIM), jnp.float32)
    v2 = jnp.zeros((1, OUT_DIM), jnp.float32)
    for ch in range(OUT_DIM // CW):
        v1 = v1 + jnp.dot(c1[ch:ch + 1, :], wbt_ref[ch],
                          preferred_element_type=jnp.float32)
        v2 = v2 + jnp.dot(c2[ch:ch + 1, :], wbt_ref[ch],
                          preferred_element_type=jnp.float32)
    v_ref[0:1, :] = v1
    v_ref[1:2, :] = v2


def _bilinear_vecs(s1, s2, wbt3):
    return pl.pallas_call(
        _bilinear_body,
        out_shape=jax.ShapeDtypeStruct((2, OUT_DIM), jnp.float32),
    )(s1, s2, wbt3)


def _logits_body(aggb_ref, aggn_ref, din_ref, v_ref, b1_ref, b2_ref,
                 a1_ref, a2_ref, bb_ref, out_ref):
    din = lax.rsqrt(jnp.maximum(din_ref[...], 1.0))
    a1 = a1_ref[0, 0]
    a2 = a2_ref[0, 0]
    bb = bb_ref[0, 0]
    hb = jnp.concatenate([aggb_ref[ch] for ch in range(NCH)], axis=1)
    hn = jnp.concatenate([aggn_ref[ch] for ch in range(NCH)], axis=1)
    x1 = hb[:, :OUT_DIM] * din + b1_ref[...]
    h1 = jnp.where(x1 > 0, x1, a1 * x1)
    x3 = hb[:, OUT_DIM:] * din + b1_ref[...]
    h3 = jnp.where(x3 > 0, x3, a1 * x3)
    x2 = hn[:, :OUT_DIM] + b2_ref[...]
    h2 = jnp.where(x2 > 0, x2, a2 * x2)
    x4 = hn[:, OUT_DIM:] + b2_ref[...]
    h4 = jnp.where(x4 > 0, x4, a2 * x4)
    dn = (((1,), (1,)), ((), ()))
    d2 = lax.dot_general(h2, v_ref[...], dn, preferred_element_type=jnp.float32)
    d1 = lax.dot_general(h1, v_ref[...], dn, preferred_element_type=jnp.float32)
    d4 = lax.dot_general(h4, v_ref[...], dn, preferred_element_type=jnp.float32)
    d3 = lax.dot_general(h3, v_ref[...], dn, preferred_element_type=jnp.float32)
    out_ref[...] = jnp.concatenate(
        [d2[:, 0:1], d1[:, 1:2], d4[:, 0:1], d3[:, 1:2]], axis=1) + bb


def _logits(aggb, aggn, din_col, v, b1f, b2f, a1s, a2s, bbs):
    bm = RPT
    return pl.pallas_call(
        _logits_body,
        grid=(NPAD // bm,),
        in_specs=[
            pl.BlockSpec((NCH, bm, CW), lambda i: (0, i, 0)),
            pl.BlockSpec((NCH, bm, CW), lambda i: (0, i, 0)),
            pl.BlockSpec((bm, 1), lambda i: (i, 0)),
            pl.BlockSpec((2, OUT_DIM), lambda i: (0, 0)),
            pl.BlockSpec((1, OUT_DIM), lambda i: (0, 0)),
            pl.BlockSpec((1, OUT_DIM), lambda i: (0, 0)),
            pl.BlockSpec(memory_space=pltpu.SMEM),
            pl.BlockSpec(memory_space=pltpu.SMEM),
            pl.BlockSpec(memory_space=pltpu.SMEM),
        ],
        out_specs=pl.BlockSpec((bm, 4), lambda i: (i, 0)),
        out_shape=jax.ShapeDtypeStruct((NPAD, 4), jnp.float32),
    )(aggb, aggn, din_col, v, b1f, b2f, a1s, a2s, bbs)


def kernel(feat, shuf_feat, edge_index, diff_edge_index, edge_weight,
           W1, b1, a1, W2, b2, a2, Wb, bb):
    zhbm = jnp.zeros((NPAD, CW), jnp.float32)
    deg = _degree_kernel(edge_index, zhbm)
    dout_col = deg[0, :N, 0:1]
    din_col = deg[1, :, 0:1]

    tb, tn = _make_tables(feat, shuf_feat, W1, W2, dout_col)
    eicat = jnp.transpose(edge_index.reshape(2, NBAT, KB),
                          (1, 0, 2)).reshape(NBAT * 2 * KB)
    deicat = jnp.transpose(diff_edge_index.reshape(2, NBAT, KB),
                           (1, 0, 2)).reshape(NBAT * 2 * KB)
    aggb, aggn = _agg_kernel(tb, tn, eicat, deicat, edge_weight, zhbm)

    b1r = b1.reshape(OUT_DIM // CW, CW)
    b2r = b2.reshape(OUT_DIM // CW, CW)
    a1s = a1.reshape(1, 1)
    a2s = a2.reshape(1, 1)
    bbs = bb.reshape(1, 1)
    s1, s2 = _col_sums(aggb, aggn, din_col, b1r, b2r, a1s, a2s)
    v = _bilinear_vecs(s1, s2, Wb.T.reshape(OUT_DIM // CW, CW, OUT_DIM))
    out = _logits(aggb, aggn, din_col, v, b1.reshape(1, OUT_DIM),
                  b2.reshape(1, OUT_DIM), a1s, a2s, bbs)
    return out[:N].T.reshape(4 * N)
